# baseline (speedup 1.0000x reference)
import functools

import jax
import jax.numpy as jnp
from jax import lax
from jax.experimental import pallas as pl
from jax.experimental.pallas import tpu as pltpu
from jax.experimental.pallas import tpu_sc as plsc

Y_LOW = 0.0
Y_HIGH = 10.5
L = 16


def _make_sc_kernel(B, NU, NM, NC, NS):
    NW = NC * NS
    bpw = B // NW
    ngroups = bpw // L
    flat = bpw * L
    mesh = plsc.VectorSubcoreMesh(core_axis_name="c", subcore_axis_name="s")

    @functools.partial(
        pl.kernel,
        mesh=mesh,
        compiler_params=pltpu.CompilerParams(use_tc_tiling_on_sc=False),
        out_type=jax.ShapeDtypeStruct((B,), jnp.float32),
        scratch_types=[
            pltpu.VMEM((bpw,), jnp.int32),
            pltpu.VMEM((bpw,), jnp.int32),
            pltpu.VMEM((flat,), jnp.int32),
            pltpu.VMEM((flat,), jnp.int32),
            pltpu.VMEM((flat,), jnp.float32),
            pltpu.VMEM((flat,), jnp.float32),
            pltpu.VMEM((bpw,), jnp.float32),
            pltpu.VMEM((bpw,), jnp.float32),
            pltpu.VMEM((bpw,), jnp.float32),
            pltpu.SemaphoreType.DMA,
        ],
    )
    def sc_kernel(uid_hbm, mid_hbm, uf_hbm, ub_hbm, mf_hbm, mb_hbm, out_hbm,
                  uid_v, mid_v, uidx, midx, ufc, mfc, ubr, mbr, outv, sem):
        sid = lax.axis_index("s")
        wid = sid * NC + lax.axis_index("c")
        base = wid * bpw
        pltpu.sync_copy(uid_hbm.at[pl.ds(base, bpw)], uid_v)
        pltpu.sync_copy(mid_hbm.at[pl.ds(base, bpw)], mid_v)
        c3 = pltpu.async_copy(ub_hbm.at[uid_v], ubr, sem)
        c4 = pltpu.async_copy(mb_hbm.at[mid_v], mbr, sem)

        for g in range(ngroups):
            sl = pl.ds(g * L, L)
            u = uid_v[sl]
            m = mid_v[sl]
            for k in range(L):
                uidx[pl.ds(k * bpw + g * L, L)] = u + (k * NU)
                midx[pl.ds(k * bpw + g * L, L)] = m + (k * NM)

        c1 = pltpu.async_copy(uf_hbm.at[uidx], ufc, sem)
        c2 = pltpu.async_copy(mf_hbm.at[midx], mfc, sem)
        c1.wait()
        c2.wait()
        c3.wait()
        c4.wait()

        for g in range(ngroups):
            sl = pl.ds(g * L, L)
            acc = ubr[sl] + mbr[sl]
            for k in range(L):
                skl = pl.ds(k * bpw + g * L, L)
                acc = acc + ufc[skl] * mfc[skl]
            outv[sl] = Y_HIGH / (1.0 + jnp.exp(-acc)) + Y_LOW

        pltpu.sync_copy(outv, out_hbm.at[pl.ds(base, bpw)])

    return sc_kernel


def kernel(x, user_factors, user_bias, movie_factors, movie_bias):
    B = x.shape[0]
    NU = user_factors.shape[0]
    NM = movie_factors.shape[0]
    info = plsc.get_sparse_core_info()
    sc_kernel = _make_sc_kernel(B, NU, NM, info.num_cores, info.num_subcores)
    uid = x[:, 0]
    mid = x[:, 1]
    uf_flat = user_factors.T.reshape(-1)
    mf_flat = movie_factors.T.reshape(-1)
    return sc_kernel(uid, mid, uf_flat, user_bias, mf_flat, movie_bias)


# speedup vs baseline: 2.5966x; 2.5966x over previous
import functools

import jax
import jax.numpy as jnp
from jax import lax
from jax.experimental import pallas as pl
from jax.experimental.pallas import tpu as pltpu
from jax.experimental.pallas import tpu_sc as plsc

Y_LOW = 0.0
Y_HIGH = 10.5
L = 16


def _make_sc_kernel(B, NU, NM, NC, NS):
    NW = NC * NS
    bpw = B // NW
    ngroups = bpw // L
    flat = bpw * L
    mesh = plsc.VectorSubcoreMesh(core_axis_name="c", subcore_axis_name="s")

    @functools.partial(
        pl.kernel,
        mesh=mesh,
        compiler_params=pltpu.CompilerParams(use_tc_tiling_on_sc=False),
        out_type=jax.ShapeDtypeStruct((B,), jnp.float32),
        scratch_types=[
            pltpu.VMEM((bpw,), jnp.int32),
            pltpu.VMEM((bpw,), jnp.int32),
            pltpu.VMEM((flat,), jnp.int32),
            pltpu.VMEM((flat,), jnp.int32),
            pltpu.VMEM((flat,), jnp.float32),
            pltpu.VMEM((flat,), jnp.float32),
            pltpu.VMEM((bpw,), jnp.float32),
            pltpu.VMEM((bpw,), jnp.float32),
            pltpu.VMEM((bpw,), jnp.float32),
            pltpu.SemaphoreType.DMA,
        ],
    )
    def sc_kernel(uid_hbm, mid_hbm, uf_hbm, ub_hbm, mf_hbm, mb_hbm, out_hbm,
                  uid_v, mid_v, uidx, midx, ufc, mfc, ubr, mbr, outv, sem):
        sid = lax.axis_index("s")
        wid = sid * NC + lax.axis_index("c")
        base = wid * bpw
        pltpu.sync_copy(uid_hbm.at[pl.ds(base, bpw)], uid_v)
        pltpu.sync_copy(mid_hbm.at[pl.ds(base, bpw)], mid_v)
        c3 = pltpu.async_copy(ub_hbm.at[uid_v], ubr, sem)
        c4 = pltpu.async_copy(mb_hbm.at[mid_v], mbr, sem)

        for g in range(ngroups):
            sl = pl.ds(g * L, L)
            u = uid_v[sl]
            m = mid_v[sl]
            u = u * L
            m = m * L
            for k in range(L):
                uidx[pl.ds(k * bpw + g * L, L)] = u + k
                midx[pl.ds(k * bpw + g * L, L)] = m + k

        c1 = pltpu.async_copy(uf_hbm.at[uidx], ufc, sem)
        c2 = pltpu.async_copy(mf_hbm.at[midx], mfc, sem)
        c1.wait()
        c2.wait()
        c3.wait()
        c4.wait()

        for g in range(ngroups):
            sl = pl.ds(g * L, L)
            acc = ubr[sl] + mbr[sl]
            for k in range(L):
                skl = pl.ds(k * bpw + g * L, L)
                acc = acc + ufc[skl] * mfc[skl]
            outv[sl] = Y_HIGH / (1.0 + jnp.exp(-acc)) + Y_LOW

        pltpu.sync_copy(outv, out_hbm.at[pl.ds(base, bpw)])

    return sc_kernel


def kernel(x, user_factors, user_bias, movie_factors, movie_bias):
    B = x.shape[0]
    NU = user_factors.shape[0]
    NM = movie_factors.shape[0]
    info = plsc.get_sparse_core_info()
    sc_kernel = _make_sc_kernel(B, NU, NM, info.num_cores, info.num_subcores)
    uid = x[:, 0]
    mid = x[:, 1]
    uf_flat = user_factors.reshape(-1)
    mf_flat = movie_factors.reshape(-1)
    return sc_kernel(uid, mid, uf_flat, user_bias, mf_flat, movie_bias)


# speedup vs baseline: 2.6871x; 1.0349x over previous
import functools

import jax
import jax.numpy as jnp
from jax import lax
from jax.experimental import pallas as pl
from jax.experimental.pallas import tpu as pltpu
from jax.experimental.pallas import tpu_sc as plsc

Y_LOW = 0.0
Y_HIGH = 10.5
L = 16


def _make_sc_kernel(B, NC, NS):
    NW = NC * NS
    bpw = B // NW
    ngroups = bpw // L
    flat = bpw * L
    mesh = plsc.VectorSubcoreMesh(core_axis_name="c", subcore_axis_name="s")

    @functools.partial(
        pl.kernel,
        mesh=mesh,
        compiler_params=pltpu.CompilerParams(use_tc_tiling_on_sc=False),
        out_type=jax.ShapeDtypeStruct((B,), jnp.float32),
        scratch_types=[
            pltpu.VMEM((bpw,), jnp.int32),
            pltpu.VMEM((bpw,), jnp.int32),
            pltpu.VMEM((bpw, L), jnp.float32),
            pltpu.VMEM((bpw, L), jnp.float32),
            pltpu.VMEM((bpw,), jnp.float32),
            pltpu.VMEM((bpw,), jnp.float32),
            pltpu.VMEM((flat + L,), jnp.float32),
            pltpu.VMEM((bpw,), jnp.int32),
            pltpu.VMEM((bpw,), jnp.float32),
            pltpu.VMEM((bpw,), jnp.float32),
            pltpu.VMEM_SHARED((NS * bpw * L,), jnp.float32),
            pltpu.SemaphoreType.DMA,
        ],
    )
    def sc_kernel(uid_hbm, mid_hbm, uf_hbm, ub_hbm, mf_hbm, mb_hbm, out_hbm,
                  uid_v, mid_v, ufr, mfr, ubr, mbr, prodf, idxg, accv, outv,
                  shf, sem):
        sid = lax.axis_index("s")
        wid = sid * NC + lax.axis_index("c")
        base = wid * bpw
        pltpu.sync_copy(uid_hbm.at[pl.ds(base, bpw)], uid_v)
        pltpu.sync_copy(mid_hbm.at[pl.ds(base, bpw)], mid_v)
        c1 = pltpu.async_copy(uf_hbm.at[uid_v], ufr, sem)
        c2 = pltpu.async_copy(mf_hbm.at[mid_v], mfr, sem)
        c3 = pltpu.async_copy(ub_hbm.at[uid_v], ubr, sem)
        c4 = pltpu.async_copy(mb_hbm.at[mid_v], mbr, sem)

        iota16 = lax.iota(jnp.int32, L)
        sbase = sid * flat
        for g in range(ngroups):
            idxg[pl.ds(g * L, L)] = iota16 * L + (sbase + g * L * L)

        c1.wait()
        c2.wait()
        c3.wait()
        c4.wait()

        for i in range(bpw):
            prodf[pl.ds(i * L, L)] = ufr[i] * mfr[i]

        def tree_pass(d):
            def body(c, _):
                b0 = c * (8 * L)
                for u in range(8):
                    off = b0 + u * L
                    v = prodf[pl.ds(off, L)] + prodf[pl.ds(off + d, L)]
                    prodf[pl.ds(off, L)] = v
                return 0

            lax.fori_loop(0, bpw // 8, body, 0)

        tree_pass(8)
        tree_pass(4)
        tree_pass(2)
        tree_pass(1)

        pltpu.sync_copy(prodf.at[pl.ds(0, flat)], shf.at[pl.ds(sbase, flat)])
        pltpu.async_copy(shf.at[idxg], accv, sem).wait()

        for g in range(ngroups):
            sl = pl.ds(g * L, L)
            acc = accv[sl] + ubr[sl] + mbr[sl]
            outv[sl] = Y_HIGH / (1.0 + jnp.exp(-acc)) + Y_LOW

        pltpu.sync_copy(outv, out_hbm.at[pl.ds(base, bpw)])

    return sc_kernel


def kernel(x, user_factors, user_bias, movie_factors, movie_bias):
    B = x.shape[0]
    info = plsc.get_sparse_core_info()
    sc_kernel = _make_sc_kernel(B, info.num_cores, info.num_subcores)
    uid = x[:, 0]
    mid = x[:, 1]
    return sc_kernel(uid, mid, user_factors, user_bias,
                     movie_factors, movie_bias)
